# baseline (speedup 1.0000x reference)
import functools

import jax
import jax.numpy as jnp
from jax import lax
from jax.experimental import pallas as pl
from jax.experimental.pallas import tpu as pltpu
from jax.experimental.pallas import tpu_sc as plsc

T = 2048
D = 2048
FF = 2048
E = 8
TOPK = 2
M = T * TOPK

BM = 256
M_BLOCKS = M // BM
NUM_ITEMS = M_BLOCKS + E - 1
BF = 1024
N_FF = FF // BF


def _routing_metadata(offsets):
    b_grid = jnp.arange(M_BLOCKS, dtype=jnp.int32)[:, None]
    e_grid = jnp.arange(E, dtype=jnp.int32)[None, :]
    lo_g = jnp.maximum(offsets[:-1][None, :], b_grid * BM)
    hi_g = jnp.minimum(offsets[1:][None, :], (b_grid + 1) * BM)
    valid = (lo_g < hi_g).reshape(-1)
    pos = jnp.where(valid, jnp.cumsum(valid.astype(jnp.int32)) - 1, NUM_ITEMS)
    vals = jnp.stack([jnp.broadcast_to(b_grid, (M_BLOCKS, E)).reshape(-1),
                      jnp.broadcast_to(e_grid, (M_BLOCKS, E)).reshape(-1),
                      (lo_g - b_grid * BM).reshape(-1),
                      (hi_g - b_grid * BM).reshape(-1)], axis=1)
    defaults = jnp.broadcast_to(
        jnp.array([M_BLOCKS - 1, E - 1, 0, 0], jnp.int32), (NUM_ITEMS, 4))
    items = defaults.at[pos].set(vals.astype(jnp.int32), mode="drop")
    b_arr, e_arr, lo_arr, hi_arr = (items[:, 0], items[:, 1], items[:, 2],
                                    items[:, 3])
    first = ((lo_arr == 0) & (hi_arr > 0)).astype(jnp.int32)
    return b_arr, e_arr, lo_arr, hi_arr, first


_NC = 2
_NS = 16
_NW = _NC * _NS
_LANES = 16

_GPW = M // _NW
_GCH = 32
_TPW = T // _NW
_CCH = 16


def _sc_mesh():
    return plsc.VectorSubcoreMesh(core_axis_name="c", subcore_axis_name="s")


def _sc_wid():
    return lax.axis_index("s") * _NC + lax.axis_index("c")


def _sc_gather(hidden_states, gidx):
    @functools.partial(
        pl.kernel,
        mesh=_sc_mesh(),
        out_type=jax.ShapeDtypeStruct((M, D), jnp.float32),
        scratch_types=[
            pltpu.VMEM((_GPW,), jnp.int32),
            pltpu.VMEM((_GCH, D), jnp.float32),
            pltpu.SemaphoreType.DMA,
        ],
    )
    def k(hid_hbm, gidx_hbm, xs_hbm, idx_v, rows_v, sem):
        base = _sc_wid() * _GPW
        pltpu.sync_copy(gidx_hbm.at[pl.ds(base, _GPW)], idx_v)
        for c in range(_GPW // _GCH):
            pltpu.async_copy(hid_hbm.at[idx_v.at[pl.ds(c * _GCH, _GCH)]],
                             rows_v, sem).wait()
            pltpu.sync_copy(rows_v, xs_hbm.at[pl.ds(base + c * _GCH, _GCH)])

    return k(hidden_states, gidx)


def _sc_combine(ys, i0, i1):
    @functools.partial(
        pl.kernel,
        mesh=_sc_mesh(),
        out_type=jax.ShapeDtypeStruct((T, D), jnp.float32),
        scratch_types=[
            pltpu.VMEM((_TPW,), jnp.int32),
            pltpu.VMEM((_TPW,), jnp.int32),
            pltpu.VMEM((_CCH, D), jnp.float32),
            pltpu.VMEM((_CCH, D), jnp.float32),
            pltpu.SemaphoreType.DMA,
            pltpu.SemaphoreType.DMA,
        ],
    )
    def k(ys_hbm, i0_hbm, i1_hbm, out_hbm, i0_v, i1_v, a_v, b_v, sem, semb):
        base = _sc_wid() * _TPW
        pltpu.sync_copy(i0_hbm.at[pl.ds(base, _TPW)], i0_v)
        pltpu.sync_copy(i1_hbm.at[pl.ds(base, _TPW)], i1_v)
        for c in range(_TPW // _CCH):
            cpa = pltpu.async_copy(ys_hbm.at[i0_v.at[pl.ds(c * _CCH, _CCH)]],
                                   a_v, sem)
            cpb = pltpu.async_copy(ys_hbm.at[i1_v.at[pl.ds(c * _CCH, _CCH)]],
                                   b_v, semb)
            cpa.wait()
            cpb.wait()
            for r in range(_CCH):
                def body(q, _, r=r):
                    sl = pl.ds(q * _LANES, _LANES)
                    a_v[r, sl] = a_v[r, sl] + b_v[r, sl]
                    return 0
                lax.fori_loop(0, D // _LANES, body, 0, unroll=16)
            pltpu.sync_copy(a_v, out_hbm.at[pl.ds(base + c * _CCH, _CCH)])

    return k(ys, i0, i1)


def _fc1_body(b_ref, e_ref, lo_ref, hi_ref, first_ref, x_ref, w1a_ref, w1b_ref,
              h_ref):
    i = pl.program_id(1)
    lo = lo_ref[i]
    hi = hi_ref[i]
    first = first_ref[i]

    @pl.when(hi > lo)
    def _():
        x = x_ref[...].astype(jnp.bfloat16)
        w1a = w1a_ref[0].astype(jnp.bfloat16)
        w1b = w1b_ref[0].astype(jnp.bfloat16)
        p = jnp.dot(x, w1a, preferred_element_type=jnp.float32)
        g = jnp.dot(x, w1b, preferred_element_type=jnp.float32)
        val = jax.nn.silu(p) * g
        rows = lax.broadcasted_iota(jnp.int32, (BM, BF), 0)
        val = jnp.where((rows >= lo) & (rows < hi), val, 0.0).astype(jnp.bfloat16)

        @pl.when(first == 1)
        def _():
            h_ref[...] = val

        @pl.when(first == 0)
        def _():
            h_ref[...] += val


def _fc2_body(b_ref, e_ref, lo_ref, hi_ref, first_ref, h_ref, w2_ref, s_ref,
              y_ref):
    i = pl.program_id(0)
    lo = lo_ref[i]
    hi = hi_ref[i]
    first = first_ref[i]

    @pl.when(hi > lo)
    def _():
        w2 = w2_ref[0].astype(jnp.bfloat16)
        y = jnp.dot(h_ref[...], w2, preferred_element_type=jnp.float32)
        y = y * s_ref[...]
        rows = lax.broadcasted_iota(jnp.int32, (BM, D), 0)
        val = jnp.where((rows >= lo) & (rows < hi), y, 0.0)

        @pl.when(first == 1)
        def _():
            y_ref[...] = val

        @pl.when(first == 0)
        def _():
            y_ref[...] += val


def _grouped_mlp(meta, xs, W1, W2, s_sorted, interpret=False):
    b_arr, e_arr, lo_arr, hi_arr, first = meta
    fc1 = pl.pallas_call(
        _fc1_body,
        grid_spec=pltpu.PrefetchScalarGridSpec(
            num_scalar_prefetch=5,
            grid=(N_FF, NUM_ITEMS),
            in_specs=[
                pl.BlockSpec((BM, D), lambda j, i, b, e, lo, hi, fs: (b[i], 0)),
                pl.BlockSpec((1, D, BF),
                             lambda j, i, b, e, lo, hi, fs: (e[i], 0, j)),
                pl.BlockSpec((1, D, BF),
                             lambda j, i, b, e, lo, hi, fs: (e[i], 0, N_FF + j)),
            ],
            out_specs=pl.BlockSpec((BM, BF),
                                   lambda j, i, b, e, lo, hi, fs: (b[i], j)),
        ),
        out_shape=jax.ShapeDtypeStruct((M, FF), jnp.bfloat16),
        interpret=interpret,
    )
    h = fc1(b_arr, e_arr, lo_arr, hi_arr, first, xs, W1, W1)
    fc2 = pl.pallas_call(
        _fc2_body,
        grid_spec=pltpu.PrefetchScalarGridSpec(
            num_scalar_prefetch=5,
            grid=(NUM_ITEMS,),
            in_specs=[
                pl.BlockSpec((BM, FF), lambda i, b, e, lo, hi, fs: (b[i], 0)),
                pl.BlockSpec((1, FF, D), lambda i, b, e, lo, hi, fs: (e[i], 0, 0)),
                pl.BlockSpec((BM, 1), lambda i, b, e, lo, hi, fs: (b[i], 0)),
            ],
            out_specs=pl.BlockSpec((BM, D),
                                   lambda i, b, e, lo, hi, fs: (b[i], 0)),
        ),
        out_shape=jax.ShapeDtypeStruct((M, D), jnp.float32),
        interpret=interpret,
    )
    return fc2(b_arr, e_arr, lo_arr, hi_arr, first, h, W2, s_sorted)


def kernel(hidden_states, router_logits, W1, W2):
    top_logits, top_indices = lax.top_k(router_logits, TOPK)
    scores = jax.nn.softmax(top_logits, axis=-1)
    flat = top_indices.reshape(-1).astype(jnp.int32)

    oh = (flat[:, None] == jnp.arange(E, dtype=jnp.int32)[None, :])
    oh = oh.astype(jnp.int32)
    ranks = jnp.cumsum(oh, axis=0) - oh
    counts = jnp.sum(oh, axis=0)
    offsets = jnp.concatenate([jnp.zeros((1,), jnp.int32),
                               jnp.cumsum(counts).astype(jnp.int32)])
    inv = (offsets[flat]
           + jnp.take_along_axis(ranks, flat[:, None], axis=1)[:, 0])
    iota_m = jnp.arange(M, dtype=jnp.int32)
    sorted_idx = jnp.zeros((M,), jnp.int32).at[inv].set(iota_m, unique_indices=True)
    meta = _routing_metadata(offsets)

    xs = _sc_gather(hidden_states, sorted_idx // TOPK)
    s_sorted = jnp.zeros((M,), jnp.float32).at[inv].set(
        scores.reshape(-1), unique_indices=True)

    ys = _grouped_mlp(meta, xs, W1, W2, s_sorted[:, None])

    inv2 = inv.reshape(T, TOPK)
    return _sc_combine(ys, inv2[:, 0], inv2[:, 1])


# speedup vs baseline: 1.0214x; 1.0214x over previous
import functools

import jax
import jax.numpy as jnp
from jax import lax
from jax.experimental import pallas as pl
from jax.experimental.pallas import tpu as pltpu
from jax.experimental.pallas import tpu_sc as plsc

T = 2048
D = 2048
FF = 2048
E = 8
TOPK = 2
M = T * TOPK

BM = 256
M_BLOCKS = M // BM
NUM_ITEMS = M_BLOCKS + E - 1
BF = 1024
N_FF = FF // BF


def _routing_metadata(offsets):
    b_grid = jnp.arange(M_BLOCKS, dtype=jnp.int32)[:, None]
    e_grid = jnp.arange(E, dtype=jnp.int32)[None, :]
    lo_g = jnp.maximum(offsets[:-1][None, :], b_grid * BM)
    hi_g = jnp.minimum(offsets[1:][None, :], (b_grid + 1) * BM)
    valid = (lo_g < hi_g).reshape(-1)
    pos = jnp.where(valid, jnp.cumsum(valid.astype(jnp.int32)) - 1, NUM_ITEMS)
    vals = jnp.stack([jnp.broadcast_to(b_grid, (M_BLOCKS, E)).reshape(-1),
                      jnp.broadcast_to(e_grid, (M_BLOCKS, E)).reshape(-1),
                      (lo_g - b_grid * BM).reshape(-1),
                      (hi_g - b_grid * BM).reshape(-1)], axis=1)
    defaults = jnp.broadcast_to(
        jnp.array([M_BLOCKS - 1, E - 1, 0, 0], jnp.int32), (NUM_ITEMS, 4))
    items = defaults.at[pos].set(vals.astype(jnp.int32), mode="drop")
    b_arr, e_arr, lo_arr, hi_arr = (items[:, 0], items[:, 1], items[:, 2],
                                    items[:, 3])
    first = ((lo_arr == 0) & (hi_arr > 0)).astype(jnp.int32)
    return b_arr, e_arr, lo_arr, hi_arr, first


_NC = 2
_NS = 16
_NW = _NC * _NS
_LANES = 16

_GPW = M // _NW
_GCH = 16
_TPW = T // _NW
_CCH = 16


def _sc_mesh():
    return plsc.VectorSubcoreMesh(core_axis_name="c", subcore_axis_name="s")


def _sc_wid():
    return lax.axis_index("s") * _NC + lax.axis_index("c")


def _sc_gather(hidden_states, gidx):
    @functools.partial(
        pl.kernel,
        mesh=_sc_mesh(),
        out_type=jax.ShapeDtypeStruct((M, D), jnp.float32),
        scratch_types=[
            pltpu.VMEM((_GPW,), jnp.int32),
            pltpu.VMEM((_GCH, D), jnp.float32),
            pltpu.VMEM((_GCH, D), jnp.float32),
            pltpu.SemaphoreType.DMA,
            pltpu.SemaphoreType.DMA,
        ],
    )
    def k(hid_hbm, gidx_hbm, xs_hbm, idx_v, rows_a, rows_b, sem_a, sem_b):
        base = _sc_wid() * _GPW
        pltpu.sync_copy(gidx_hbm.at[pl.ds(base, _GPW)], idx_v)
        bufs = (rows_a, rows_b)
        sems = (sem_a, sem_b)
        n = _GPW // _GCH

        def start(c):
            return pltpu.async_copy(
                hid_hbm.at[idx_v.at[pl.ds(c * _GCH, _GCH)]],
                bufs[c % 2], sems[c % 2])

        cps = {0: start(0)}
        for c in range(n):
            if c + 1 < n:
                cps[c + 1] = start(c + 1)
            cps[c].wait()
            pltpu.sync_copy(bufs[c % 2],
                            xs_hbm.at[pl.ds(base + c * _GCH, _GCH)])

    return k(hidden_states, gidx)


def _sc_combine(ys, i0, i1):
    @functools.partial(
        pl.kernel,
        mesh=_sc_mesh(),
        out_type=jax.ShapeDtypeStruct((T, D), jnp.float32),
        scratch_types=[
            pltpu.VMEM((_TPW,), jnp.int32),
            pltpu.VMEM((_TPW,), jnp.int32),
            pltpu.VMEM((_CCH, D), jnp.float32),
            pltpu.VMEM((_CCH, D), jnp.float32),
            pltpu.SemaphoreType.DMA,
            pltpu.SemaphoreType.DMA,
        ],
    )
    def k(ys_hbm, i0_hbm, i1_hbm, out_hbm, i0_v, i1_v, a_v, b_v, sem, semb):
        base = _sc_wid() * _TPW
        pltpu.sync_copy(i0_hbm.at[pl.ds(base, _TPW)], i0_v)
        pltpu.sync_copy(i1_hbm.at[pl.ds(base, _TPW)], i1_v)
        for c in range(_TPW // _CCH):
            cpa = pltpu.async_copy(ys_hbm.at[i0_v.at[pl.ds(c * _CCH, _CCH)]],
                                   a_v, sem)
            cpb = pltpu.async_copy(ys_hbm.at[i1_v.at[pl.ds(c * _CCH, _CCH)]],
                                   b_v, semb)
            cpa.wait()
            cpb.wait()
            for r in range(_CCH):
                def body(q, _, r=r):
                    sl = pl.ds(q * _LANES, _LANES)
                    a_v[r, sl] = a_v[r, sl] + b_v[r, sl]
                    return 0
                lax.fori_loop(0, D // _LANES, body, 0, unroll=16)
            pltpu.sync_copy(a_v, out_hbm.at[pl.ds(base + c * _CCH, _CCH)])

    return k(ys, i0, i1)


def _fc1_body(b_ref, e_ref, lo_ref, hi_ref, first_ref, x_ref, w1a_ref, w1b_ref,
              h_ref):
    i = pl.program_id(1)
    lo = lo_ref[i]
    hi = hi_ref[i]
    first = first_ref[i]

    @pl.when(hi > lo)
    def _():
        x = x_ref[...].astype(jnp.bfloat16)
        w1a = w1a_ref[0].astype(jnp.bfloat16)
        w1b = w1b_ref[0].astype(jnp.bfloat16)
        p = jnp.dot(x, w1a, preferred_element_type=jnp.float32)
        g = jnp.dot(x, w1b, preferred_element_type=jnp.float32)
        val = jax.nn.silu(p) * g
        rows = lax.broadcasted_iota(jnp.int32, (BM, BF), 0)
        val = jnp.where((rows >= lo) & (rows < hi), val, 0.0).astype(jnp.bfloat16)

        @pl.when(first == 1)
        def _():
            h_ref[...] = val

        @pl.when(first == 0)
        def _():
            h_ref[...] += val


def _fc2_body(b_ref, e_ref, lo_ref, hi_ref, first_ref, h_ref, w2_ref, s_ref,
              y_ref):
    i = pl.program_id(0)
    lo = lo_ref[i]
    hi = hi_ref[i]
    first = first_ref[i]

    @pl.when(hi > lo)
    def _():
        w2 = w2_ref[0].astype(jnp.bfloat16)
        y = jnp.dot(h_ref[...], w2, preferred_element_type=jnp.float32)
        y = y * s_ref[...]
        rows = lax.broadcasted_iota(jnp.int32, (BM, D), 0)
        val = jnp.where((rows >= lo) & (rows < hi), y, 0.0)

        @pl.when(first == 1)
        def _():
            y_ref[...] = val

        @pl.when(first == 0)
        def _():
            y_ref[...] += val


def _grouped_mlp(meta, xs, W1, W2, s_sorted, interpret=False):
    b_arr, e_arr, lo_arr, hi_arr, first = meta
    fc1 = pl.pallas_call(
        _fc1_body,
        grid_spec=pltpu.PrefetchScalarGridSpec(
            num_scalar_prefetch=5,
            grid=(N_FF, NUM_ITEMS),
            in_specs=[
                pl.BlockSpec((BM, D), lambda j, i, b, e, lo, hi, fs: (b[i], 0)),
                pl.BlockSpec((1, D, BF),
                             lambda j, i, b, e, lo, hi, fs: (e[i], 0, j)),
                pl.BlockSpec((1, D, BF),
                             lambda j, i, b, e, lo, hi, fs: (e[i], 0, N_FF + j)),
            ],
            out_specs=pl.BlockSpec((BM, BF),
                                   lambda j, i, b, e, lo, hi, fs: (b[i], j)),
        ),
        out_shape=jax.ShapeDtypeStruct((M, FF), jnp.bfloat16),
        interpret=interpret,
    )
    h = fc1(b_arr, e_arr, lo_arr, hi_arr, first, xs, W1, W1)
    fc2 = pl.pallas_call(
        _fc2_body,
        grid_spec=pltpu.PrefetchScalarGridSpec(
            num_scalar_prefetch=5,
            grid=(NUM_ITEMS,),
            in_specs=[
                pl.BlockSpec((BM, FF), lambda i, b, e, lo, hi, fs: (b[i], 0)),
                pl.BlockSpec((1, FF, D), lambda i, b, e, lo, hi, fs: (e[i], 0, 0)),
                pl.BlockSpec((BM, 1), lambda i, b, e, lo, hi, fs: (b[i], 0)),
            ],
            out_specs=pl.BlockSpec((BM, D),
                                   lambda i, b, e, lo, hi, fs: (b[i], 0)),
        ),
        out_shape=jax.ShapeDtypeStruct((M, D), jnp.float32),
        interpret=interpret,
    )
    return fc2(b_arr, e_arr, lo_arr, hi_arr, first, h, W2, s_sorted)


def kernel(hidden_states, router_logits, W1, W2):
    top_logits, top_indices = lax.top_k(router_logits, TOPK)
    scores = jax.nn.softmax(top_logits, axis=-1)
    flat = top_indices.reshape(-1).astype(jnp.int32)

    sorted_idx = jnp.argsort(flat, stable=True).astype(jnp.int32)
    counts = jnp.bincount(flat, length=E)
    offsets = jnp.concatenate([jnp.zeros((1,), jnp.int32),
                               jnp.cumsum(counts).astype(jnp.int32)])
    inv = jnp.zeros((M,), jnp.int32).at[sorted_idx].set(
        jnp.arange(M, dtype=jnp.int32), unique_indices=True)
    meta = _routing_metadata(offsets)

    xs = _sc_gather(hidden_states, sorted_idx // TOPK)
    s_sorted = scores.reshape(-1)[sorted_idx]

    ys = _grouped_mlp(meta, xs, W1, W2, s_sorted[:, None])

    inv2 = inv.reshape(T, TOPK)
    return _sc_combine(ys, inv2[:, 0], inv2[:, 1])
